# Initial kernel scaffold; baseline (speedup 1.0000x reference)
import jax
import jax.numpy as jnp
from jax.experimental import pallas as pl


def kernel(x, edge_attr, edge_index, graph_ids, W_d0, b_d0, W_e0, b_e0, W_p0, b_p0, W_d1, b_d1, W_e1, b_e1, W_p1, b_p1, gamma, beta, W_out, b_out):
    raise NotImplementedError("write your pallas kernel here")


# speedup vs baseline: 2.6052x; 2.6052x over previous
import functools

import jax
import jax.numpy as jnp
from jax import lax
from jax.experimental import pallas as pl
from jax.experimental.pallas import tpu as pltpu
from jax.experimental.pallas import tpu_sc as plsc

N = 10000
E = 320000
D = 128
G = 256

NC = 2
NS = 16
NW = NC * NS
EPW = E // NW
C = 80
NCHUNK = EPW // C
NP = 10240
SR = NP // NS
ZR = 128
LANES = 16


def _mish(v):
    return v * jnp.tanh(jax.nn.softplus(v))


def _node_dense_body(x_ref, w_ref, b_ref, o_ref):
    h = jnp.dot(x_ref[...], w_ref[...], preferred_element_type=jnp.float32)
    o_ref[...] = _mish(h + b_ref[...][None, :])


def _node_dense(x, w, b, bn=2000):
    n = x.shape[0]
    return pl.pallas_call(
        _node_dense_body,
        grid=(n // bn,),
        in_specs=[
            pl.BlockSpec((bn, D), lambda i: (i, 0)),
            pl.BlockSpec((D, D), lambda i: (0, 0)),
            pl.BlockSpec((D,), lambda i: (0,)),
        ],
        out_specs=pl.BlockSpec((bn, D), lambda i: (i, 0)),
        out_shape=jax.ShapeDtypeStruct((n, D), jnp.float32),
    )(x, w, b)


def _edge_dense_body(ea_ref, w0_ref, b0_ref, w1_ref, b1_ref, o0_ref, o1_ref):
    ea = ea_ref[...]
    h0 = jnp.dot(ea, w0_ref[...], preferred_element_type=jnp.float32)
    o0_ref[...] = _mish(h0 + b0_ref[...][None, :])
    h1 = jnp.dot(ea, w1_ref[...], preferred_element_type=jnp.float32)
    o1_ref[...] = _mish(h1 + b1_ref[...][None, :])


def _edge_dense(ea, w0, b0, w1, b1, be=3200):
    wspec = pl.BlockSpec((D, D), lambda i: (0, 0))
    bspec = pl.BlockSpec((D,), lambda i: (0,))
    return pl.pallas_call(
        _edge_dense_body,
        grid=(E // be,),
        in_specs=[pl.BlockSpec((be, D), lambda i: (i, 0)), wspec, bspec, wspec, bspec],
        out_specs=[pl.BlockSpec((be, D), lambda i: (i, 0))] * 2,
        out_shape=[jax.ShapeDtypeStruct((E, D), jnp.float32)] * 2,
    )(ea, w0, b0, w1, b1)


def _sc_gine(hx, he, src, dst):
    mesh = plsc.VectorSubcoreMesh(core_axis_name="c", subcore_axis_name="s")

    @functools.partial(
        pl.kernel,
        out_type=jax.ShapeDtypeStruct((NC, NP, D), jnp.float32),
        mesh=mesh,
        scratch_types=[
            pltpu.VMEM((C,), jnp.int32),
            pltpu.VMEM((C,), jnp.int32),
            pltpu.VMEM((C, D), jnp.float32),
            pltpu.VMEM((C, D), jnp.float32),
            pltpu.VMEM((ZR, D), jnp.float32),
            pltpu.VMEM_SHARED((NP, D), jnp.float32),
            pltpu.SemaphoreType.DMA,
        ],
    )
    def k(hx_hbm, he_hbm, src_hbm, dst_hbm, out_hbm, sidx, didx, hxg, msg, zbuf, agg, sem):
        c = lax.axis_index("c")
        s = lax.axis_index("s")
        wid = s * NC + c

        def zrow(r, carry):
            for i in range(D // LANES):
                zbuf[r, pl.ds(i * LANES, LANES)] = jnp.zeros((LANES,), jnp.float32)
            return carry

        lax.fori_loop(0, ZR, zrow, 0)
        base_row = s * SR
        for j in range(SR // ZR):
            pltpu.sync_copy(zbuf, agg.at[pl.ds(base_row + j * ZR, ZR)])
        plsc.subcore_barrier()

        ebase = wid * EPW

        def chunk(j, carry):
            off = ebase + j * C
            pltpu.sync_copy(src_hbm.at[pl.ds(off, C)], sidx)
            pltpu.sync_copy(dst_hbm.at[pl.ds(off, C)], didx)
            pltpu.async_copy(hx_hbm.at[sidx], hxg, sem).wait()
            pltpu.sync_copy(he_hbm.at[pl.ds(off, C)], msg)

            def row(r, rc):
                for i in range(D // LANES):
                    sl = pl.ds(i * LANES, LANES)
                    msg[r, sl] = jnp.maximum(hxg[r, sl] + msg[r, sl], 0.0)
                return rc

            lax.fori_loop(0, C, row, 0)
            pltpu.sync_copy(msg, agg.at[didx], add=True)
            return carry

        lax.fori_loop(0, NCHUNK, chunk, 0)
        plsc.subcore_barrier()

        for j in range(SR // ZR):
            r0 = base_row + j * ZR
            pltpu.sync_copy(agg.at[pl.ds(r0, ZR)], zbuf)
            pltpu.sync_copy(zbuf, out_hbm.at[c, pl.ds(r0, ZR)])

    return k(hx, he, src, dst)[:, :N, :]


def _combine_body(hx_ref, agg_ref, wp_ref, bp_ref, wd_ref, bd_ref, o_ref):
    out = hx_ref[...] + agg_ref[0] + agg_ref[1]
    h = _mish(jnp.dot(out, wp_ref[...], preferred_element_type=jnp.float32)
              + bp_ref[...][None, :])
    o_ref[...] = _mish(jnp.dot(h, wd_ref[...], preferred_element_type=jnp.float32)
                       + bd_ref[...][None, :])


def _combine(hx, agg, wp, bp, wd, bd, bn=2000):
    wspec = pl.BlockSpec((D, D), lambda i: (0, 0))
    bspec = pl.BlockSpec((D,), lambda i: (0,))
    return pl.pallas_call(
        _combine_body,
        grid=(N // bn,),
        in_specs=[
            pl.BlockSpec((bn, D), lambda i: (i, 0)),
            pl.BlockSpec((NC, bn, D), lambda i: (0, i, 0)),
            wspec, bspec, wspec, bspec,
        ],
        out_specs=pl.BlockSpec((bn, D), lambda i: (i, 0)),
        out_shape=jax.ShapeDtypeStruct((N, D), jnp.float32),
    )(hx, agg, wp, bp, wd, bd)


def _final_body(hx_ref, agg_ref, gid_ref, wp_ref, bp_ref, gamma_ref, beta_ref,
                wout_ref, bout_ref, o_ref, sums_ref, counts_ref, bn):
    step = pl.program_id(0)

    @pl.when(step == 0)
    def _init():
        sums_ref[...] = jnp.zeros_like(sums_ref)
        counts_ref[...] = jnp.zeros_like(counts_ref)

    out = hx_ref[...] + agg_ref[0] + agg_ref[1]
    h = _mish(jnp.dot(out, wp_ref[...], preferred_element_type=jnp.float32)
              + bp_ref[...][None, :])
    gid = gid_ref[0]
    giota = lax.broadcasted_iota(jnp.int32, (G, bn), 0)
    onehot = (gid == giota).astype(jnp.float32)
    sums_ref[...] += jnp.dot(onehot, h, preferred_element_type=jnp.float32,
                             precision=lax.Precision.HIGHEST)
    counts_ref[...] += jnp.sum(onehot, axis=1, keepdims=True)

    @pl.when(step == pl.num_programs(0) - 1)
    def _fin():
        pooled = sums_ref[...] / jnp.maximum(counts_ref[...], 1.0)
        mu = jnp.mean(pooled, axis=0, keepdims=True)
        var = jnp.mean(jnp.square(pooled - mu), axis=0, keepdims=True)
        xn = (pooled - mu) * lax.rsqrt(var + 1e-5)
        xn = xn * gamma_ref[...][None, :] + beta_ref[...][None, :]
        o_ref[...] = jnp.dot(xn, wout_ref[...], preferred_element_type=jnp.float32) \
            + bout_ref[...][None, :]


def _final(hx, agg, gid3, wp, bp, gamma, beta, wout, bout, bn=2000):
    wspec = pl.BlockSpec((D, D), lambda i: (0, 0))
    bspec = pl.BlockSpec((D,), lambda i: (0,))
    return pl.pallas_call(
        functools.partial(_final_body, bn=bn),
        grid=(N // bn,),
        in_specs=[
            pl.BlockSpec((bn, D), lambda i: (i, 0)),
            pl.BlockSpec((NC, bn, D), lambda i: (0, i, 0)),
            pl.BlockSpec((1, 1, bn), lambda i: (i, 0, 0)),
            wspec, bspec, bspec, bspec,
            pl.BlockSpec((D, 1), lambda i: (0, 0)),
            pl.BlockSpec((1,), lambda i: (0,)),
        ],
        out_specs=pl.BlockSpec((G, 1), lambda i: (0, 0)),
        out_shape=jax.ShapeDtypeStruct((G, 1), jnp.float32),
        scratch_shapes=[
            pltpu.VMEM((G, D), jnp.float32),
            pltpu.VMEM((G, 1), jnp.float32),
        ],
    )(hx, agg, gid3, wp, bp, gamma, beta, wout, bout)


def kernel(x, edge_attr, edge_index, graph_ids,
           W_d0, b_d0, W_e0, b_e0, W_p0, b_p0,
           W_d1, b_d1, W_e1, b_e1, W_p1, b_p1,
           gamma, beta, W_out, b_out):
    src = edge_index[0]
    dst = edge_index[1]
    bn = 2000
    gid3 = graph_ids.reshape(N // bn, 1, bn)

    hx0 = _node_dense(x, W_d0, b_d0, bn=bn)
    he0, he1 = _edge_dense(edge_attr, W_e0, b_e0, W_e1, b_e1)
    agg0 = _sc_gine(hx0, he0, src, dst)
    hx1 = _combine(hx0, agg0, W_p0, b_p0, W_d1, b_d1, bn=bn)
    agg1 = _sc_gine(hx1, he1, src, dst)
    return _final(hx1, agg1, gid3, W_p1, b_p1, gamma, beta, W_out, b_out, bn=bn)


# speedup vs baseline: 4.8525x; 1.8626x over previous
import functools

import jax
import jax.numpy as jnp
from jax import lax
from jax.experimental import pallas as pl
from jax.experimental.pallas import tpu as pltpu
from jax.experimental.pallas import tpu_sc as plsc

N = 10000
E = 320000
D = 128
G = 256

NC = 2
NS = 16
NW = NC * NS
EPW = E // NW
C = 40
NCHUNK = EPW // C
NP = 10240
SR = NP // NS
ZR = 32
LANES = 16


def _mish(v):
    return v * jnp.tanh(jax.nn.softplus(v))


def _node_dense_body(x_ref, w_ref, b_ref, o_ref):
    h = jnp.dot(x_ref[...], w_ref[...], preferred_element_type=jnp.float32)
    o_ref[...] = _mish(h + b_ref[...][None, :])


def _node_dense(x, w, b, bn=2000):
    n = x.shape[0]
    return pl.pallas_call(
        _node_dense_body,
        grid=(n // bn,),
        in_specs=[
            pl.BlockSpec((bn, D), lambda i: (i, 0)),
            pl.BlockSpec((D, D), lambda i: (0, 0)),
            pl.BlockSpec((D,), lambda i: (0,)),
        ],
        out_specs=pl.BlockSpec((bn, D), lambda i: (i, 0)),
        out_shape=jax.ShapeDtypeStruct((n, D), jnp.float32),
    )(x, w, b)


def _edge_dense_body(ea_ref, w0_ref, b0_ref, w1_ref, b1_ref, o0_ref, o1_ref):
    ea = ea_ref[...]
    h0 = jnp.dot(ea, w0_ref[...], preferred_element_type=jnp.float32)
    o0_ref[...] = _mish(h0 + b0_ref[...][None, :])
    h1 = jnp.dot(ea, w1_ref[...], preferred_element_type=jnp.float32)
    o1_ref[...] = _mish(h1 + b1_ref[...][None, :])


def _edge_dense(ea, w0, b0, w1, b1, be=3200):
    wspec = pl.BlockSpec((D, D), lambda i: (0, 0))
    bspec = pl.BlockSpec((D,), lambda i: (0,))
    return pl.pallas_call(
        _edge_dense_body,
        grid=(E // be,),
        in_specs=[pl.BlockSpec((be, D), lambda i: (i, 0)), wspec, bspec, wspec, bspec],
        out_specs=[pl.BlockSpec((be, D), lambda i: (i, 0))] * 2,
        out_shape=[jax.ShapeDtypeStruct((E, D), jnp.float32)] * 2,
    )(ea, w0, b0, w1, b1)


def _sc_gine(hx, he, src, dst):
    mesh = plsc.VectorSubcoreMesh(core_axis_name="c", subcore_axis_name="s")
    NB = 4

    @functools.partial(
        pl.kernel,
        out_type=jax.ShapeDtypeStruct((NC, NP, D), jnp.float32),
        mesh=mesh,
        scratch_types=[
            pltpu.VMEM((NB, C), jnp.int32),
            pltpu.VMEM((NB, C), jnp.int32),
            pltpu.VMEM((NB, C, D), jnp.float32),
            pltpu.VMEM((NB, C, D), jnp.float32),
            pltpu.VMEM((ZR, D), jnp.float32),
            pltpu.VMEM_SHARED((NP, D), jnp.float32),
            pltpu.SemaphoreType.DMA((NB,)),
            pltpu.SemaphoreType.DMA((NB,)),
            pltpu.SemaphoreType.DMA((NB,)),
        ],
    )
    def k(hx_hbm, he_hbm, src_hbm, dst_hbm, out_hbm,
          sidx, didx, hxg, msg, zbuf, agg, sem_i, sem_d, sem_s):
        c = lax.axis_index("c")
        s = lax.axis_index("s")
        wid = s * NC + c

        def zrow(r, carry):
            for i in range(D // LANES):
                zbuf[r, pl.ds(i * LANES, LANES)] = jnp.zeros((LANES,), jnp.float32)
            return carry

        lax.fori_loop(0, ZR, zrow, 0)
        base_row = s * SR
        for j in range(SR // ZR):
            pltpu.sync_copy(zbuf, agg.at[pl.ds(base_row + j * ZR, ZR)])
        plsc.subcore_barrier()

        ebase = wid * EPW

        def start_idx(j, b):
            off = ebase + j * C
            pltpu.async_copy(src_hbm.at[pl.ds(off, C)], sidx.at[b], sem_i.at[b])
            pltpu.async_copy(dst_hbm.at[pl.ds(off, C)], didx.at[b], sem_i.at[b])

        def wait_idx(b):
            pltpu.make_async_copy(src_hbm.at[pl.ds(0, C)], sidx.at[b], sem_i.at[b]).wait()
            pltpu.make_async_copy(dst_hbm.at[pl.ds(0, C)], didx.at[b], sem_i.at[b]).wait()

        def start_dat(j, b):
            off = ebase + j * C
            pltpu.async_copy(hx_hbm.at[sidx.at[b]], hxg.at[b], sem_d.at[b])
            pltpu.async_copy(he_hbm.at[pl.ds(off, C)], msg.at[b], sem_d.at[b])

        def wait_dat(b):
            pltpu.make_async_copy(he_hbm.at[pl.ds(0, C)], hxg.at[b], sem_d.at[b]).wait()
            pltpu.make_async_copy(he_hbm.at[pl.ds(0, C)], msg.at[b], sem_d.at[b]).wait()

        def start_sc(b):
            pltpu.async_copy(msg.at[b], agg.at[didx.at[b]], sem_s.at[b], add=True)

        def wait_sc(b):
            pltpu.make_async_copy(msg.at[b], agg.at[didx.at[b]], sem_s.at[b]).wait()

        start_idx(0, 0)
        start_idx(1, 1)
        wait_idx(0)
        start_dat(0, 0)

        def outer(g, carry):
            for b in range(NB):
                j = g * NB + b
                b2 = (b + 2) % NB

                @pl.when(j + 2 < NCHUNK)
                def _a():
                    @pl.when(j >= NB - 2)
                    def _w():
                        wait_sc(b2)
                    start_idx(j + 2, b2)

                b1 = (b + 1) % NB

                @pl.when(j + 1 < NCHUNK)
                def _b():
                    wait_idx(b1)
                    start_dat(j + 1, b1)

                @pl.when(j < NCHUNK)
                def _c():
                    wait_dat(b)

                    def row(r, rc):
                        for i in range(D // LANES):
                            sl = pl.ds(i * LANES, LANES)
                            msg[b, r, sl] = jnp.maximum(hxg[b, r, sl] + msg[b, r, sl], 0.0)
                        return rc

                    lax.fori_loop(0, C, row, 0)
                    start_sc(b)
            return carry

        lax.fori_loop(0, (NCHUNK + NB - 1) // NB, outer, 0)
        for b in range(NB):
            wait_sc(b)
        plsc.subcore_barrier()

        for j in range(SR // ZR):
            r0 = base_row + j * ZR
            pltpu.sync_copy(agg.at[pl.ds(r0, ZR)], zbuf)
            pltpu.sync_copy(zbuf, out_hbm.at[c, pl.ds(r0, ZR)])

    return k(hx, he, src, dst)[:, :N, :]


def _combine_body(hx_ref, agg_ref, wp_ref, bp_ref, wd_ref, bd_ref, o_ref):
    out = hx_ref[...] + agg_ref[0] + agg_ref[1]
    h = _mish(jnp.dot(out, wp_ref[...], preferred_element_type=jnp.float32)
              + bp_ref[...][None, :])
    o_ref[...] = _mish(jnp.dot(h, wd_ref[...], preferred_element_type=jnp.float32)
                       + bd_ref[...][None, :])


def _combine(hx, agg, wp, bp, wd, bd, bn=2000):
    wspec = pl.BlockSpec((D, D), lambda i: (0, 0))
    bspec = pl.BlockSpec((D,), lambda i: (0,))
    return pl.pallas_call(
        _combine_body,
        grid=(N // bn,),
        in_specs=[
            pl.BlockSpec((bn, D), lambda i: (i, 0)),
            pl.BlockSpec((NC, bn, D), lambda i: (0, i, 0)),
            wspec, bspec, wspec, bspec,
        ],
        out_specs=pl.BlockSpec((bn, D), lambda i: (i, 0)),
        out_shape=jax.ShapeDtypeStruct((N, D), jnp.float32),
    )(hx, agg, wp, bp, wd, bd)


def _final_body(hx_ref, agg_ref, gid_ref, wp_ref, bp_ref, gamma_ref, beta_ref,
                wout_ref, bout_ref, o_ref, sums_ref, counts_ref, bn):
    step = pl.program_id(0)

    @pl.when(step == 0)
    def _init():
        sums_ref[...] = jnp.zeros_like(sums_ref)
        counts_ref[...] = jnp.zeros_like(counts_ref)

    out = hx_ref[...] + agg_ref[0] + agg_ref[1]
    h = _mish(jnp.dot(out, wp_ref[...], preferred_element_type=jnp.float32)
              + bp_ref[...][None, :])
    gid = gid_ref[0]
    giota = lax.broadcasted_iota(jnp.int32, (G, bn), 0)
    onehot = (gid == giota).astype(jnp.float32)
    sums_ref[...] += jnp.dot(onehot, h, preferred_element_type=jnp.float32,
                             precision=lax.Precision.HIGHEST)
    counts_ref[...] += jnp.sum(onehot, axis=1, keepdims=True)

    @pl.when(step == pl.num_programs(0) - 1)
    def _fin():
        pooled = sums_ref[...] / jnp.maximum(counts_ref[...], 1.0)
        mu = jnp.mean(pooled, axis=0, keepdims=True)
        var = jnp.mean(jnp.square(pooled - mu), axis=0, keepdims=True)
        xn = (pooled - mu) * lax.rsqrt(var + 1e-5)
        xn = xn * gamma_ref[...][None, :] + beta_ref[...][None, :]
        o_ref[...] = jnp.dot(xn, wout_ref[...], preferred_element_type=jnp.float32) \
            + bout_ref[...][None, :]


def _final(hx, agg, gid3, wp, bp, gamma, beta, wout, bout, bn=2000):
    wspec = pl.BlockSpec((D, D), lambda i: (0, 0))
    bspec = pl.BlockSpec((D,), lambda i: (0,))
    return pl.pallas_call(
        functools.partial(_final_body, bn=bn),
        grid=(N // bn,),
        in_specs=[
            pl.BlockSpec((bn, D), lambda i: (i, 0)),
            pl.BlockSpec((NC, bn, D), lambda i: (0, i, 0)),
            pl.BlockSpec((1, 1, bn), lambda i: (i, 0, 0)),
            wspec, bspec, bspec, bspec,
            pl.BlockSpec((D, 1), lambda i: (0, 0)),
            pl.BlockSpec((1,), lambda i: (0,)),
        ],
        out_specs=pl.BlockSpec((G, 1), lambda i: (0, 0)),
        out_shape=jax.ShapeDtypeStruct((G, 1), jnp.float32),
        scratch_shapes=[
            pltpu.VMEM((G, D), jnp.float32),
            pltpu.VMEM((G, 1), jnp.float32),
        ],
    )(hx, agg, gid3, wp, bp, gamma, beta, wout, bout)


def kernel(x, edge_attr, edge_index, graph_ids,
           W_d0, b_d0, W_e0, b_e0, W_p0, b_p0,
           W_d1, b_d1, W_e1, b_e1, W_p1, b_p1,
           gamma, beta, W_out, b_out):
    src = edge_index[0]
    dst = edge_index[1]
    bn = 2000
    gid3 = graph_ids.reshape(N // bn, 1, bn)

    hx0 = _node_dense(x, W_d0, b_d0, bn=bn)
    he0, he1 = _edge_dense(edge_attr, W_e0, b_e0, W_e1, b_e1)
    agg0 = _sc_gine(hx0, he0, src, dst)
    hx1 = _combine(hx0, agg0, W_p0, b_p0, W_d1, b_d1, bn=bn)
    agg1 = _sc_gine(hx1, he1, src, dst)
    return _final(hx1, agg1, gid3, W_p1, b_p1, gamma, beta, W_out, b_out, bn=bn)
